# baseline (speedup 1.0000x reference)
import functools

import jax
import jax.numpy as jnp
from jax import lax
from jax.experimental import pallas as pl
from jax.experimental.pallas import tpu as pltpu
from jax.experimental.pallas import tpu_sc as plsc

NC = 2
NS = 16
NW = NC * NS
L = 16
CH = 256


def _sc_body(per_w, rpb, x_hbm, u_hbm, m_hbm, out_hbm,
             xv, blk_u, blk_m, sub_u, sub_m, rows_u, rows_m, outv,
             sem_x, sem_u, sem_m):
    wid = lax.axis_index("s") * NC + lax.axis_index("c")
    base = wid * per_w
    sub_mask = rpb - 1
    sub_shift = 3

    pltpu.async_copy(x_hbm.at[pl.ds(2 * base, 2 * per_w)], xv, sem_x).wait()

    iota = lax.iota(jnp.int32, L)

    @pl.loop(0, per_w, step=L)
    def _(i):
        rows2 = (iota + i) * 2
        users = plsc.load_gather(xv, [rows2])
        movies = plsc.load_gather(xv, [rows2 + 1])
        blk_u.at[pl.ds(i, L)][...] = lax.shift_right_logical(users, sub_shift)
        blk_m.at[pl.ds(i, L)][...] = lax.shift_right_logical(movies, sub_shift)
        sub_u.at[pl.ds(i, L)][...] = (users & sub_mask) * L
        sub_m.at[pl.ds(i, L)][...] = (movies & sub_mask) * L

    for c in range(per_w // CH):
        cu = pltpu.async_copy(u_hbm.at[blk_u.at[pl.ds(c * CH, CH)]],
                              rows_u, sem_u)
        cm = pltpu.async_copy(m_hbm.at[blk_m.at[pl.ds(c * CH, CH)]],
                              rows_m, sem_m)
        cu.wait()
        cm.wait()

        @pl.loop(0, CH, step=L)
        def _(r0):
            i0 = c * CH + r0
            su = sub_u.at[pl.ds(i0, L)][...]
            sm = sub_m.at[pl.ds(i0, L)][...]
            acc = jnp.zeros((L,), jnp.float32)
            for j in range(L):
                offu = pl.multiple_of(su[j], L)
                offm = pl.multiple_of(sm[j], L)
                u = rows_u.at[r0 + j, pl.ds(offu, L)][...]
                m = rows_m.at[r0 + j, pl.ds(offm, L)][...]
                acc = jnp.where(iota == j, jnp.sum(u * m), acc)
            outv.at[pl.ds(i0, L)][...] = acc

    pltpu.sync_copy(outv, out_hbm.at[pl.ds(base, per_w)])


def kernel(x, U, M):
    batch = x.shape[0]
    per_w = batch // NW
    n, d = U.shape
    rpb = 128 // d
    nb = n // rpb

    u_r = U.reshape(nb, 128)
    m_r = M.reshape(nb, 128)
    x_f = x.reshape(-1)

    mesh = plsc.VectorSubcoreMesh(core_axis_name="c", subcore_axis_name="s")
    cp = pltpu.CompilerParams(needs_layout_passes=False)
    k = pl.kernel(
        functools.partial(_sc_body, per_w, rpb),
        out_type=jax.ShapeDtypeStruct((batch,), jnp.float32),
        mesh=mesh,
        scratch_types=[
            pltpu.VMEM((2 * per_w,), jnp.int32),
            pltpu.VMEM((per_w,), jnp.int32),
            pltpu.VMEM((per_w,), jnp.int32),
            pltpu.VMEM((per_w,), jnp.int32),
            pltpu.VMEM((per_w,), jnp.int32),
            pltpu.VMEM((CH, 128), jnp.float32),
            pltpu.VMEM((CH, 128), jnp.float32),
            pltpu.VMEM((per_w,), jnp.float32),
            pltpu.SemaphoreType.DMA,
            pltpu.SemaphoreType.DMA,
            pltpu.SemaphoreType.DMA,
        ],
        compiler_params=cp,
    )
    out = k(x_f, u_r, m_r)
    return out.reshape(-1, 1)


# speedup vs baseline: 5.3629x; 5.3629x over previous
import functools

import jax
import jax.numpy as jnp
from jax import lax
from jax.experimental import pallas as pl
from jax.experimental.pallas import tpu as pltpu
from jax.experimental.pallas import tpu_sc as plsc

NC = 2
NS = 16
NW = NC * NS
L = 16
K = 16


def _sc_body(per_w, x_hbm, ut_hbm, mt_hbm, out_hbm,
             xv, tiles_u, tiles_m, outv,
             sem_x, sem_u, sem_m):
    wid = lax.axis_index("s") * NC + lax.axis_index("c")
    base = wid * per_w

    pltpu.async_copy(x_hbm.at[pl.ds(2 * base, 2 * per_w)], xv, sem_x).wait()

    iota = lax.iota(jnp.int32, L)

    @pl.loop(0, per_w, step=K)
    def _(i):
        rows2 = (iota + i) * 2
        uvec = plsc.load_gather(xv, [rows2])
        mvec = plsc.load_gather(xv, [rows2 + 1])
        cu_all = lax.shift_right_logical(uvec, 7) * 128
        cm_all = lax.shift_right_logical(mvec, 7) * 128
        ru_all = uvec & 127
        rm_all = mvec & 127

        for j in range(K):
            cu = pl.multiple_of(cu_all[j], 128)
            cm = pl.multiple_of(cm_all[j], 128)
            pltpu.async_copy(ut_hbm.at[:, pl.ds(cu, 128)],
                             tiles_u.at[j], sem_u)
            pltpu.async_copy(mt_hbm.at[:, pl.ds(cm, 128)],
                             tiles_m.at[j], sem_m)
        for j in range(K):
            pltpu.make_async_copy(ut_hbm.at[:, pl.ds(0, 128)],
                                  tiles_u.at[j], sem_u).wait()
            pltpu.make_async_copy(mt_hbm.at[:, pl.ds(0, 128)],
                                  tiles_m.at[j], sem_m).wait()

        acc = jnp.zeros((L,), jnp.float32)
        for j in range(K):
            ju = jnp.full((L,), j, jnp.int32)
            ru = ru_all[j] + jnp.zeros((L,), jnp.int32)
            rm = rm_all[j] + jnp.zeros((L,), jnp.int32)
            u = plsc.load_gather(tiles_u, [ju, iota, ru])
            m = plsc.load_gather(tiles_m, [ju, iota, rm])
            acc = jnp.where(iota == j, jnp.sum(u * m), acc)
        outv.at[pl.ds(i, L)][...] = acc

    pltpu.sync_copy(outv, out_hbm.at[pl.ds(base, per_w)])


def kernel(x, U, M):
    batch = x.shape[0]
    per_w = batch // NW
    dim = U.shape[1]

    ut = U.T
    mt = M.T
    x_f = x.reshape(-1)

    mesh = plsc.VectorSubcoreMesh(core_axis_name="c", subcore_axis_name="s")
    cp = pltpu.CompilerParams(needs_layout_passes=False)
    k = pl.kernel(
        functools.partial(_sc_body, per_w),
        out_type=jax.ShapeDtypeStruct((batch,), jnp.float32),
        mesh=mesh,
        scratch_types=[
            pltpu.VMEM((2 * per_w,), jnp.int32),
            pltpu.VMEM((K, dim, 128), jnp.float32),
            pltpu.VMEM((K, dim, 128), jnp.float32),
            pltpu.VMEM((per_w,), jnp.float32),
            pltpu.SemaphoreType.DMA,
            pltpu.SemaphoreType.DMA,
            pltpu.SemaphoreType.DMA,
        ],
        compiler_params=cp,
    )
    out = k(x_f, ut, mt)
    return out.reshape(-1, 1)


# speedup vs baseline: 5.9175x; 1.1034x over previous
import functools

import jax
import jax.numpy as jnp
from jax import lax
from jax.experimental import pallas as pl
from jax.experimental.pallas import tpu as pltpu
from jax.experimental.pallas import tpu_sc as plsc

NC = 2
NS = 16
NW = NC * NS
L = 16
HK = 8


def _sc_body(per_w, x_hbm, ut_hbm, mt_hbm, out_hbm,
             xv, tiles_u, tiles_m, outv,
             sem_x, su0, su1, sm0, sm1):
    wid = lax.axis_index("s") * NC + lax.axis_index("c")
    base = wid * per_w

    pltpu.async_copy(x_hbm.at[pl.ds(2 * base, 2 * per_w)], xv, sem_x).wait()

    iota = lax.iota(jnp.int32, L)

    def load_idx(i):
        rows2 = (iota + i) * 2
        return plsc.load_gather(xv, [rows2]), plsc.load_gather(xv, [rows2 + 1])

    def fire(uvec, mvec, lane0, slot, sem_u, sem_m):
        cu_all = lax.shift_right_logical(uvec, 7) * 128
        cm_all = lax.shift_right_logical(mvec, 7) * 128
        for jj in range(HK):
            j = lane0 + jj
            cu = pl.multiple_of(cu_all[j], 128)
            cm = pl.multiple_of(cm_all[j], 128)
            pltpu.async_copy(ut_hbm.at[:, pl.ds(cu, 128)],
                             tiles_u.at[slot * HK + jj], sem_u)
            pltpu.async_copy(mt_hbm.at[:, pl.ds(cm, 128)],
                             tiles_m.at[slot * HK + jj], sem_m)

    def drain(slot, sem_u, sem_m):
        for jj in range(HK):
            pltpu.make_async_copy(ut_hbm.at[:, pl.ds(0, 128)],
                                  tiles_u.at[slot * HK + jj], sem_u).wait()
            pltpu.make_async_copy(mt_hbm.at[:, pl.ds(0, 128)],
                                  tiles_m.at[slot * HK + jj], sem_m).wait()

    def reduce_half(acc, uvec, mvec, lane0, slot):
        ru_all = uvec & 127
        rm_all = mvec & 127
        for jj in range(HK):
            j = lane0 + jj
            jv = jnp.full((L,), slot * HK + jj, jnp.int32)
            ru = ru_all[j] + jnp.zeros((L,), jnp.int32)
            rm = rm_all[j] + jnp.zeros((L,), jnp.int32)
            u = plsc.load_gather(tiles_u, [jv, iota, ru])
            m = plsc.load_gather(tiles_m, [jv, iota, rm])
            acc = jnp.where(iota == j, jnp.sum(u * m), acc)
        return acc

    uv0, mv0 = load_idx(0)
    fire(uv0, mv0, 0, 0, su0, sm0)

    @pl.loop(0, per_w, step=2 * HK)
    def _(i):
        uvec, mvec = load_idx(i)
        fire(uvec, mvec, HK, 1, su1, sm1)

        drain(0, su0, sm0)
        acc = reduce_half(jnp.zeros((L,), jnp.float32), uvec, mvec, 0, 0)

        @pl.when(i + 2 * HK < per_w)
        def _():
            uvn, mvn = load_idx(i + 2 * HK)
            fire(uvn, mvn, 0, 0, su0, sm0)

        drain(1, su1, sm1)
        acc2 = reduce_half(acc, uvec, mvec, HK, 1)
        outv.at[pl.ds(i, L)][...] = acc2

    pltpu.sync_copy(outv, out_hbm.at[pl.ds(base, per_w)])


def kernel(x, U, M):
    batch = x.shape[0]
    per_w = batch // NW
    dim = U.shape[1]

    ut = U.T
    mt = M.T
    x_f = x.reshape(-1)

    mesh = plsc.VectorSubcoreMesh(core_axis_name="c", subcore_axis_name="s")
    cp = pltpu.CompilerParams(needs_layout_passes=False)
    k = pl.kernel(
        functools.partial(_sc_body, per_w),
        out_type=jax.ShapeDtypeStruct((batch,), jnp.float32),
        mesh=mesh,
        scratch_types=[
            pltpu.VMEM((2 * per_w,), jnp.int32),
            pltpu.VMEM((2 * HK, dim, 128), jnp.float32),
            pltpu.VMEM((2 * HK, dim, 128), jnp.float32),
            pltpu.VMEM((per_w,), jnp.float32),
            pltpu.SemaphoreType.DMA,
            pltpu.SemaphoreType.DMA,
            pltpu.SemaphoreType.DMA,
            pltpu.SemaphoreType.DMA,
            pltpu.SemaphoreType.DMA,
        ],
        compiler_params=cp,
    )
    out = k(x_f, ut, mt)
    return out.reshape(-1, 1)


# speedup vs baseline: 6.4208x; 1.0851x over previous
import functools

import jax
import jax.numpy as jnp
from jax import lax
from jax.experimental import pallas as pl
from jax.experimental.pallas import tpu as pltpu
from jax.experimental.pallas import tpu_sc as plsc

NC = 2
NS = 16
NW = NC * NS
L = 16
HK = 8
NSLOT = 3


def _sc_body(per_w, x_hbm, ut_hbm, mt_hbm, out_hbm,
             xv, tiles_u, tiles_m, outv,
             sem_x, *sems):
    wid = lax.axis_index("s") * NC + lax.axis_index("c")
    base = wid * per_w
    n_rounds = per_w // HK

    pltpu.async_copy(x_hbm.at[pl.ds(2 * base, 2 * per_w)], xv, sem_x).wait()

    iota = lax.iota(jnp.int32, L)
    sem_u = sems[:NSLOT]
    sem_m = sems[NSLOT:]

    def fire(r, slot):
        rows2 = (iota + r * HK) * 2
        rows2 = jnp.minimum(rows2, 2 * per_w - 2)
        uvec = plsc.load_gather(xv, [rows2])
        mvec = plsc.load_gather(xv, [rows2 + 1])
        cu_all = lax.shift_right_logical(uvec, 7) * 128
        cm_all = lax.shift_right_logical(mvec, 7) * 128
        for jj in range(HK):
            cu = pl.multiple_of(cu_all[jj], 128)
            cm = pl.multiple_of(cm_all[jj], 128)
            pltpu.async_copy(ut_hbm.at[:, pl.ds(cu, 128)],
                             tiles_u.at[slot * HK + jj], sem_u[slot])
            pltpu.async_copy(mt_hbm.at[:, pl.ds(cm, 128)],
                             tiles_m.at[slot * HK + jj], sem_m[slot])

    def drain_compute(r, slot):
        for jj in range(HK):
            pltpu.make_async_copy(ut_hbm.at[:, pl.ds(0, 128)],
                                  tiles_u.at[slot * HK + jj],
                                  sem_u[slot]).wait()
            pltpu.make_async_copy(mt_hbm.at[:, pl.ds(0, 128)],
                                  tiles_m.at[slot * HK + jj],
                                  sem_m[slot]).wait()
        rows2 = (iota + r * HK) * 2
        rows2 = jnp.minimum(rows2, 2 * per_w - 2)
        uvec = plsc.load_gather(xv, [rows2])
        mvec = plsc.load_gather(xv, [rows2 + 1])
        ru_all = uvec & 127
        rm_all = mvec & 127
        out_slot = pl.ds(pl.multiple_of(r * HK, HK), L)
        acc = outv.at[out_slot][...]
        for jj in range(HK):
            jv = jnp.full((L,), slot * HK + jj, jnp.int32)
            ru = ru_all[jj] + jnp.zeros((L,), jnp.int32)
            rm = rm_all[jj] + jnp.zeros((L,), jnp.int32)
            u = plsc.load_gather(tiles_u, [jv, iota, ru])
            m = plsc.load_gather(tiles_m, [jv, iota, rm])
            acc = jnp.where(iota == jj, jnp.sum(u * m), acc)
        outv.at[out_slot][...] = acc

    fire(0, 0)
    fire(1, 1)

    @pl.loop(0, n_rounds - 1, step=NSLOT)
    def _(r):
        fire(r + 2, 2)
        drain_compute(r, 0)
        fire(r + 3, 0)
        drain_compute(r + 1, 1)

        @pl.when(r + 4 < n_rounds)
        def _():
            fire(r + 4, 1)

        drain_compute(r + 2, 2)

    drain_compute(n_rounds - 1, 0)

    pltpu.sync_copy(outv.at[pl.ds(0, per_w)], out_hbm.at[pl.ds(base, per_w)])


def kernel(x, U, M):
    batch = x.shape[0]
    per_w = batch // NW
    dim = U.shape[1]

    ut = U.T
    mt = M.T
    x_f = x.reshape(-1)

    mesh = plsc.VectorSubcoreMesh(core_axis_name="c", subcore_axis_name="s")
    cp = pltpu.CompilerParams(needs_layout_passes=False)
    k = pl.kernel(
        functools.partial(_sc_body, per_w),
        out_type=jax.ShapeDtypeStruct((batch,), jnp.float32),
        mesh=mesh,
        scratch_types=[
            pltpu.VMEM((2 * per_w,), jnp.int32),
            pltpu.VMEM((NSLOT * HK, dim, 128), jnp.float32),
            pltpu.VMEM((NSLOT * HK, dim, 128), jnp.float32),
            pltpu.VMEM((per_w + L,), jnp.float32),
            pltpu.SemaphoreType.DMA,
            pltpu.SemaphoreType.DMA,
            pltpu.SemaphoreType.DMA,
            pltpu.SemaphoreType.DMA,
            pltpu.SemaphoreType.DMA,
            pltpu.SemaphoreType.DMA,
            pltpu.SemaphoreType.DMA,
        ],
        compiler_params=cp,
    )
    out = k(x_f, ut, mt)
    return out.reshape(-1, 1)
